# baseline (speedup 1.0000x reference)
import jax
import jax.numpy as jnp
import numpy as np
from jax import lax
from jax.experimental import pallas as pl

_B, _C, _HW = 64, 768, 196
_A = 16
_BB = 8
_J = 32
_G = _C // _J


def _head_body(x_ref, w_ref, g_ref, act_ref, sel_ref):
    v = jnp.maximum(x_ref[...], 0.0)
    parts = [
        jnp.sum(v[:, :, j * _HW:(j + 1) * _HW], axis=2) for j in range(_J)
    ]
    pooled = jnp.concatenate(parts, axis=1) * (1.0 / _HW)
    logits = lax.dot_general(
        pooled, w_ref[...], (((1,), (1,)), ((), ())),
        preferred_element_type=jnp.float32,
        precision=lax.Precision.HIGHEST)
    m = jnp.max(logits, axis=1, keepdims=True)
    e = jnp.exp(logits - m)
    p = e / jnp.sum(e, axis=1, keepdims=True)
    idx = lax.broadcasted_iota(jnp.int32, p.shape, 1)
    cand = jnp.where(p >= jnp.max(p, axis=1, keepdims=True), idx, _A)
    act = jnp.min(cand, axis=1, keepdims=True)
    act_ref[...] = act
    g = g_ref[...]
    sel = jnp.broadcast_to(g[0][None, :], (v.shape[0], g.shape[1]))
    for a in range(1, _A):
        sel = jnp.where(act == a, g[a][None, :], sel)
    sel_ref[...] = sel


_PERM = np.array([(k % _G) * _J + (k // _G) for k in range(_C)], dtype=np.int32)


def kernel(x, fc1_weight, channel_gates):
    xr = x.reshape(_B, _G, _J * _HW)
    w_perm = fc1_weight[:, _PERM]
    actions2d, selected = pl.pallas_call(
        _head_body,
        grid=(_B // _BB,),
        in_specs=[
            pl.BlockSpec((_BB, _G, _J * _HW), lambda i: (i, 0, 0)),
            pl.BlockSpec((_A, _C), lambda i: (0, 0)),
            pl.BlockSpec((_A, _C), lambda i: (0, 0)),
        ],
        out_specs=[
            pl.BlockSpec((_BB, 1), lambda i: (i, 0)),
            pl.BlockSpec((_BB, _C), lambda i: (i, 0)),
        ],
        out_shape=[
            jax.ShapeDtypeStruct((_B, 1), jnp.int32),
            jax.ShapeDtypeStruct((_B, _C), jnp.float32),
        ],
    )(xr, w_perm, channel_gates)
    return actions2d.reshape(_B), selected


# speedup vs baseline: 11.9996x; 11.9996x over previous
import jax
import jax.numpy as jnp
from jax import lax
from jax.experimental import pallas as pl
from jax.experimental.pallas import tpu as pltpu

_B, _C, _HW = 64, 768, 196
_A = 16
_K = 28
_S = _HW // _K


def _head_body(x_ref, wt_ref, g_ref, act_ref, sel_ref, acc_ref):
    i = pl.program_id(0)
    part = jnp.sum(jnp.maximum(x_ref[...], 0.0), axis=0)

    @pl.when(i == 0)
    def _():
        acc_ref[...] = part

    @pl.when(i > 0)
    def _():
        acc_ref[...] += part

    @pl.when(i == _S - 1)
    def _():
        pooled = acc_ref[...] * (1.0 / _HW)
        logits = lax.dot_general(
            pooled, wt_ref[...], (((1,), (0,)), ((), ())),
            preferred_element_type=jnp.float32,
            precision=lax.Precision.HIGHEST)
        m = jnp.max(logits, axis=1, keepdims=True)
        e = jnp.exp(logits - m)
        p = e / jnp.sum(e, axis=1, keepdims=True)
        idx = lax.broadcasted_iota(jnp.int32, p.shape, 1)
        cand = jnp.where(p >= jnp.max(p, axis=1, keepdims=True), idx, _A)
        act = jnp.min(cand, axis=1, keepdims=True)
        act_ref[...] = act
        g = g_ref[...]
        sel = jnp.broadcast_to(g[0][None, :], (_B, _C))
        for a in range(1, _A):
            sel = jnp.where(act == a, g[a][None, :], sel)
        sel_ref[...] = sel


def kernel(x, fc1_weight, channel_gates):
    xt = jnp.transpose(x, (2, 3, 0, 1)).reshape(_HW, _B, _C)
    wt = fc1_weight.T
    actions2d, selected = pl.pallas_call(
        _head_body,
        grid=(_S,),
        in_specs=[
            pl.BlockSpec((_K, _B, _C), lambda i: (i, 0, 0)),
            pl.BlockSpec((_C, _A), lambda i: (0, 0)),
            pl.BlockSpec((_A, _C), lambda i: (0, 0)),
        ],
        out_specs=[
            pl.BlockSpec((_B, 1), lambda i: (0, 0)),
            pl.BlockSpec((_B, _C), lambda i: (0, 0)),
        ],
        out_shape=[
            jax.ShapeDtypeStruct((_B, 1), jnp.int32),
            jax.ShapeDtypeStruct((_B, _C), jnp.float32),
        ],
        scratch_shapes=[pltpu.VMEM((_B, _C), jnp.float32)],
    )(xt, wt, channel_gates)
    return actions2d.reshape(_B), selected
